# baseline (speedup 1.0000x reference)
import jax
import jax.numpy as jnp
from jax.experimental import pallas as pl
from jax.experimental.pallas import tpu as pltpu


def _bisect_steps(a, k, lo, hi, steps):
    for _ in range(steps):
        mid = jnp.floor((lo + hi + 1.0) * 0.5)
        cnt = jnp.sum((a >= mid).astype(jnp.float32), axis=-1, keepdims=True)
        ge = cnt >= k
        lo = jnp.where(ge, mid, lo)
        hi = jnp.where(ge, hi, mid - 1.0)
    return lo, hi


def _bisect_finish(a, k, lo, hi):
    def cond(carry):
        lo, hi = carry
        return jnp.max(hi - lo) > 0.0

    def body(carry):
        return _bisect_steps(a, k, *carry, steps=2)

    lo, _ = jax.lax.while_loop(cond, body, (lo, hi))
    return lo


def _row_min_max(a):
    n = a.shape[-1]
    mn = a[:, :128]
    mx = mn
    for c in range(128, n, 128):
        blk = a[:, c : c + 128]
        mn = jnp.minimum(mn, blk)
        mx = jnp.maximum(mx, blk)
    return (
        jnp.min(mn, axis=-1, keepdims=True),
        jnp.max(mx, axis=-1, keepdims=True),
    )


def _phase(x_ref, wxh_ref, wcat_ref, y_ref, cat_ref, wbuf, rbuf):
    NX = x_ref.shape[1]
    half = x_ref.shape[0] // 2
    rows = (slice(0, half), slice(half, 2 * half))

    x = x_ref[...].astype(jnp.bfloat16)
    cat_ref[:, :NX] = x
    a_h = [
        jnp.dot(x[r], wxh_ref[...], preferred_element_type=jnp.float32)
        for r in rows
    ]
    hi_h = [jnp.max(a, axis=-1, keepdims=True) for a in a_h]
    pre_h = [
        _bisect_steps(a, 13, jnp.zeros_like(hi), hi, steps=4)
        for a, hi in zip(a_h, hi_h)
    ]
    thr_h = [_bisect_finish(a, 13, lo, hi) for a, (lo, hi) in zip(a_h, pre_h)]
    for p, r in enumerate(rows):
        cat_ref[r, NX:] = jnp.where(a_h[p] >= thr_h[p], a_h[p], 0.0).astype(
            jnp.bfloat16
        )

    a_y = [
        jnp.dot(cat_ref[r, :], wcat_ref[...], preferred_element_type=jnp.float32)
        for r in rows
    ]
    prev = [rbuf[r, :] for r in rows]
    pre_y = []
    for p in range(2):
        lo, hi = _row_min_max(prev[p])
        pre_y.append(_bisect_steps(prev[p], 51, lo, hi, steps=7))
    for p, r in enumerate(rows):
        wbuf[r, :] = a_y[p]

    for p, r in enumerate(rows):
        thr = _bisect_finish(prev[p], 51, *pre_y[p])
        y_ref[r, :] = jnp.where(prev[p] >= thr, prev[p], 0.0)


def _wta_block(x_ref, wxh_ref, wcat_ref, y_ref, cat_ref, buf0, buf1):
    i = pl.program_id(0)

    @pl.when(i == 0)
    def _init():
        buf1[...] = jnp.zeros_like(buf1)

    args = (x_ref, wxh_ref, wcat_ref, y_ref, cat_ref)

    @pl.when(jax.lax.rem(i, 2) == 0)
    def _even():
        _phase(*args, buf0, buf1)

    @pl.when(jax.lax.rem(i, 2) == 1)
    def _odd():
        _phase(*args, buf1, buf0)


@jax.jit
def _wta(x, w_xh, w_cat):
    B, NX = x.shape
    NH = w_xh.shape[1]
    NY = w_cat.shape[1]
    BLK = 1024
    nblk = B // BLK
    return pl.pallas_call(
        _wta_block,
        grid=(nblk + 1,),
        in_specs=[
            pl.BlockSpec((BLK, NX), lambda i: (jnp.minimum(i, nblk - 1), 0)),
            pl.BlockSpec((NX, NH), lambda i: (0, 0)),
            pl.BlockSpec((NX + NH, NY), lambda i: (0, 0)),
        ],
        out_specs=pl.BlockSpec((BLK, NY), lambda i: (jnp.maximum(i - 1, 0), 0)),
        out_shape=jax.ShapeDtypeStruct((B, NY), jnp.float32),
        scratch_shapes=[
            pltpu.VMEM((BLK, NX + NH), jnp.bfloat16),
            pltpu.VMEM((BLK, NY), jnp.float32),
            pltpu.VMEM((BLK, NY), jnp.float32),
        ],
    )(x, w_xh, w_cat)


def kernel(x, w_xy, w_xh, w_hy, k_y, k_h):
    w_cat = jnp.concatenate(
        [w_xy.astype(jnp.bfloat16), -w_hy.astype(jnp.bfloat16)], axis=0
    )
    return _wta(x, w_xh.astype(jnp.bfloat16), w_cat)


# speedup vs baseline: 1.0763x; 1.0763x over previous
import jax
import jax.numpy as jnp
from jax.experimental import pallas as pl
from jax.experimental.pallas import tpu as pltpu


def _bisect_steps(a, k, lo, hi, steps):
    for _ in range(steps):
        mid = jnp.floor((lo + hi + 1.0) * 0.5)
        cnt = jnp.sum((a >= mid).astype(jnp.float32), axis=-1, keepdims=True)
        ge = cnt >= k
        lo = jnp.where(ge, mid, lo)
        hi = jnp.where(ge, hi, mid - 1.0)
    return lo, hi


def _bisect_finish(a, k, lo, hi):
    def cond(carry):
        lo, hi = carry
        return jnp.max(hi - lo) > 0.0

    def body(carry):
        return _bisect_steps(a, k, *carry, steps=2)

    lo, _ = jax.lax.while_loop(cond, body, (lo, hi))
    return lo


def _row_min_max(a):
    n = a.shape[-1]
    mn = a[:, :128]
    mx = mn
    for c in range(128, n, 128):
        blk = a[:, c : c + 128]
        mn = jnp.minimum(mn, blk)
        mx = jnp.maximum(mx, blk)
    return (
        jnp.min(mn, axis=-1, keepdims=True),
        jnp.max(mx, axis=-1, keepdims=True),
    )


def _phase(x_ref, wxh_ref, wcat_ref, y_ref, cat_ref, wbuf, rbuf):
    NX = x_ref.shape[1]
    half = x_ref.shape[0] // 2
    rows = (slice(0, half), slice(half, 2 * half))

    x = x_ref[...].astype(jnp.bfloat16)
    cat_ref[:, :NX] = x
    a_h = [
        jnp.dot(x[r], wxh_ref[...], preferred_element_type=jnp.float32)
        for r in rows
    ]
    hi_h = [jnp.max(a, axis=-1, keepdims=True) for a in a_h]
    pre_h = [
        _bisect_steps(a, 13, jnp.zeros_like(hi), hi, steps=4)
        for a, hi in zip(a_h, hi_h)
    ]
    thr_h = [_bisect_finish(a, 13, lo, hi) for a, (lo, hi) in zip(a_h, pre_h)]
    for p, r in enumerate(rows):
        cat_ref[r, NX:] = jnp.where(a_h[p] >= thr_h[p], a_h[p], 0.0).astype(
            jnp.bfloat16
        )

    a_y = [
        jnp.dot(cat_ref[r, :], wcat_ref[...], preferred_element_type=jnp.float32)
        for r in rows
    ]
    prev = [rbuf[r, :] for r in rows]
    pre_y = []
    for p in range(2):
        lo, hi = _row_min_max(prev[p])
        pre_y.append(_bisect_steps(prev[p], 51, lo, hi, steps=6))
    for p, r in enumerate(rows):
        wbuf[r, :] = a_y[p]

    for p, r in enumerate(rows):
        thr = _bisect_finish(prev[p], 51, *pre_y[p])
        y_ref[r, :] = jnp.where(prev[p] >= thr, prev[p], 0.0)


def _wta_block(x_ref, wxh_ref, wcat_ref, y_ref, cat_ref, buf0, buf1):
    i = pl.program_id(0)

    @pl.when(i == 0)
    def _init():
        buf1[...] = jnp.zeros_like(buf1)

    args = (x_ref, wxh_ref, wcat_ref, y_ref, cat_ref)

    @pl.when(jax.lax.rem(i, 2) == 0)
    def _even():
        _phase(*args, buf0, buf1)

    @pl.when(jax.lax.rem(i, 2) == 1)
    def _odd():
        _phase(*args, buf1, buf0)


@jax.jit
def _wta(x, w_xh, w_cat):
    B, NX = x.shape
    NH = w_xh.shape[1]
    NY = w_cat.shape[1]
    BLK = 1024
    nblk = B // BLK
    return pl.pallas_call(
        _wta_block,
        grid=(nblk + 1,),
        in_specs=[
            pl.BlockSpec((BLK, NX), lambda i: (jnp.minimum(i, nblk - 1), 0)),
            pl.BlockSpec((NX, NH), lambda i: (0, 0)),
            pl.BlockSpec((NX + NH, NY), lambda i: (0, 0)),
        ],
        out_specs=pl.BlockSpec((BLK, NY), lambda i: (jnp.maximum(i - 1, 0), 0)),
        out_shape=jax.ShapeDtypeStruct((B, NY), jnp.float32),
        scratch_shapes=[
            pltpu.VMEM((BLK, NX + NH), jnp.bfloat16),
            pltpu.VMEM((BLK, NY), jnp.float32),
            pltpu.VMEM((BLK, NY), jnp.float32),
        ],
    )(x, w_xh, w_cat)


def kernel(x, w_xy, w_xh, w_hy, k_y, k_h):
    w_cat = jnp.concatenate(
        [w_xy.astype(jnp.bfloat16), -w_hy.astype(jnp.bfloat16)], axis=0
    )
    return _wta(x, w_xh.astype(jnp.bfloat16), w_cat)
